# baseline (speedup 1.0000x reference)
import jax
import jax.numpy as jnp
from jax.experimental import pallas as pl
from jax.experimental.pallas import tpu as pltpu

_V = 128
_T = 64
_B = 16
_TV = _T * _V
_NTV = float(_B * _TV)
_EPS = 1e-5
_PREC = jax.lax.Precision.DEFAULT
_F32 = jnp.float32


def _mm(a, b, dn=(((1,), (0,)), ((), ()))):
    return jax.lax.dot_general(a, b, dn, precision=_PREC,
                               preferred_element_type=_F32)


def _sumsq(v):
    return jnp.concatenate([jnp.sum(v, axis=1, keepdims=True),
                            jnp.sum(v * v, axis=1, keepdims=True)], axis=1)


def _bn2d(v, st2, g_col, b_col):
    mean = st2[:, 0:1] / _NTV
    var = st2[:, 1:2] / _NTV - mean * mean
    scale = g_col * jax.lax.rsqrt(var + _EPS)
    return v * scale + (b_col - mean * scale)


def _gcn_prep(c_in, inter, c_out, pa_ref, wa_ref, wb_ref, wdn_ref, wrs_ref,
              ba_ref, bb_ref, bdn_ref, brs_ref, wd3_ref, bd3_ref, al_ref):
    paw = pa_ref[...]
    pa_cat = jnp.concatenate([paw[i * _V:(i + 1) * _V, :]
                              for i in range(3)], axis=1)
    wfront = jnp.concatenate([wa_ref[...], wb_ref[...], wdn_ref[...],
                              wrs_ref[...]], axis=0)
    bfront = jnp.concatenate([ba_ref[...], bb_ref[...], bdn_ref[...],
                              brs_ref[...]], axis=0)
    wd3 = wd3_ref[...]
    wd = jnp.concatenate([wd3[i] for i in range(3)],
                         axis=0 if c_in == 1 else 1)
    bd_sum = jnp.transpose(jnp.sum(bd3_ref[...], axis=0, keepdims=True))
    return pa_cat, wfront, bfront, wd, bd_sum, al_ref[...]


def _gcn_stage(x2d, c_in, inter, c_out, pa_cat, wfront, bfront, wd, bd_sum,
               alpha):
    si = 6 * inter
    if c_in == 1:
        fr = wfront * x2d + bfront
    else:
        fr = _mm(wfront, x2d) + bfront
    ab = fr[:si, :]
    d = fr[si:si + c_out, :]
    r = fr[si + c_out:, :]
    ms = []
    for i in range(3):
        a_i = ab[i * inter:(i + 1) * inter, :]
        b_i = ab[(3 + i) * inter:(4 + i) * inter, :]
        a_i = a_i.reshape(inter, _T, _V).reshape(inter * _T, _V)
        b_i = b_i.reshape(inter, _T, _V).reshape(inter * _T, _V)
        m = _mm(a_i, b_i, (((0,), (0,)), ((), ())))
        ms.append(m / float(inter * _T))
    a_cat = pa_cat + jnp.tanh(jnp.concatenate(ms, axis=1)) * alpha
    if c_in == 1:
        x_tv = x2d.reshape(_T, _V)
        z_cat = _mm(x_tv, a_cat)
        y = bd_sum + jnp.zeros((c_out, _TV), _F32)
        for i in range(3):
            z2d = z_cat[:, i * _V:(i + 1) * _V].reshape(1, _TV)
            y = y + wd[i * c_out:(i + 1) * c_out, :] * z2d
    else:
        x_ctv = x2d.reshape(c_in, _T, _V).reshape(c_in * _T, _V)
        z_cat = _mm(x_ctv, a_cat)
        zr = jnp.concatenate(
            [z_cat[:, i * _V:(i + 1) * _V].reshape(c_in, _T, _V)
             .reshape(c_in, _TV) for i in range(3)], axis=0)
        y = _mm(wd, zr) + bd_sum
    return y, d, r


def _k1_body(x_ref, *refs):
    (y_ref, st_ref), w_refs = refs[-2:], refs[:-2]
    prep = _gcn_prep(1, 8, 32, *w_refs)
    for g in range(x_ref.shape[0]):
        y, d, r = _gcn_stage(x_ref[g], 1, 8, 32, *prep)
        y_ref[g] = y
        st_ref[g] = jnp.concatenate([_sumsq(y), _sumsq(d), _sumsq(r)],
                                    axis=1)


def _k3_body(tp_ref, x_ref, stt_ref, st1_ref, tg_ref, tb_ref, rg_ref,
             rb_ref, wrs_ref, brs_ref, *refs):
    (y_ref, x2_ref, st_ref), w_refs = refs[-3:], refs[:-3]
    prep = _gcn_prep(32, 16, 64, *w_refs)
    stt = jnp.sum(stt_ref[...], axis=0)
    st1 = jnp.sum(st1_ref[...], axis=0)
    for g in range(tp_ref.shape[0]):
        yt = _bn2d(tp_ref[g], stt, tg_ref[...], tb_ref[...])
        rpre = wrs_ref[...] * x_ref[g] + brs_ref[...]
        yr = _bn2d(rpre, st1[:, 4:6], rg_ref[...], rb_ref[...])
        x2d = jnp.maximum(yt + yr, 0.0)
        y, d, r = _gcn_stage(x2d, 32, 16, 64, *prep)
        y_ref[g] = y
        x2_ref[g] = x2d
        st_ref[g] = jnp.concatenate([_sumsq(y), _sumsq(d), _sumsq(r)],
                                    axis=1)


def _attn_tcn(c, c_half, g, dpre, y_ref, st_ref, gg_ref, gb_ref, dg_ref,
              db_ref, sat_ref, sab_ref, taw_ref, tab_ref, f1w_ref, f1b_ref,
              f2w_ref, f2b_ref, wt_ref, tb_ref, o_ref, ost_ref):
    st = jnp.sum(st_ref[...], axis=0)
    yb = _bn2d(y_ref[g], st[:, 0:2], gg_ref[...], gb_ref[...])
    db = _bn2d(dpre, st[:, 2:4], dg_ref[...], db_ref[...])
    y = jnp.maximum(yb + db, 0.0)
    y3 = y.reshape(c, _T, _V)
    se = jnp.mean(y3, axis=1)
    p_mat = _mm(sat_ref[...], se, (((0,), (0,)), ((), ())))
    p_big = jnp.concatenate([jnp.zeros((127, 64), _F32), p_mat,
                             jnp.zeros((127, 64), _F32)], axis=1)
    p_big = jnp.concatenate([p_big, jnp.zeros((1, 256), _F32)], axis=0)
    pr = p_big.reshape(16, 8, 256)
    b = pr[:, 0, 0:249]
    for k2 in range(1, 8):
        b = b + pr[:, k2, k2:k2 + 249]
    s1 = jnp.zeros((1, _V), _F32)
    for k1 in range(16):
        s1 = s1 + b[k1:k1 + 1, 8 * k1 + 1: 8 * k1 + 1 + _V]
    s1 = jax.nn.sigmoid(s1 + sab_ref[...])
    se2 = jnp.mean(y3 * (1.0 + s1.reshape(1, 1, _V)), axis=2)
    se2p = jnp.concatenate([jnp.zeros((c, 4), _F32), se2,
                            jnp.zeros((c, 4), _F32)], axis=1)
    taw = taw_ref[...]
    s1t = jnp.zeros((1, _T), _F32) + tab_ref[...]
    for k in range(9):
        s1t = s1t + _mm(taw[:, k:k + 1], se2p[:, k:k + _T],
                        (((0,), (0,)), ((), ())))
    s1t = jax.nn.sigmoid(s1t)
    se3 = jnp.mean(se2 * (1.0 + s1t), axis=1, keepdims=True)
    h = jnp.maximum(_mm(f1w_ref[...], se3) + f1b_ref[...], 0.0)
    s2 = jax.nn.sigmoid(_mm(f2w_ref[...], h) + f2b_ref[...])
    fac = (1.0 + jnp.transpose(s1t)) * (1.0 + s1)
    y = (y3 * fac).reshape(c, _TV)
    yp = jnp.concatenate([jnp.zeros((c, 4 * _V), _F32), y,
                          jnp.zeros((c, 4 * _V), _F32)], axis=1)
    wt = wt_ref[...] * (1.0 + jnp.transpose(s2))
    t = tb_ref[...] + jnp.zeros((c, _TV), _F32)
    for k in range(9):
        t = t + _mm(wt[k * c:(k + 1) * c, :], yp[:, k * _V:k * _V + _TV])
    o_ref[g] = t
    ost_ref[g] = _sumsq(t)


def _k2_body(y_ref, x_ref, wdn_ref, bdn_ref, *refs):
    for g in range(y_ref.shape[0]):
        dpre = wdn_ref[...] * x_ref[g] + bdn_ref[...]
        _attn_tcn(32, 16, g, dpre, y_ref, *refs)


def _k4_body(y_ref, x2_ref, wdn_ref, bdn_ref, *refs):
    for g in range(y_ref.shape[0]):
        dpre = _mm(wdn_ref[...], x2_ref[g]) + bdn_ref[...]
        _attn_tcn(64, 32, g, dpre, y_ref, *refs)


def _k5_body(tp_ref, x2_ref, stt_ref, st2_ref, tg_ref, tb_ref, rg_ref,
             rb_ref, wrs_ref, brs_ref, xc_ref, xr_ref):
    stt = jnp.sum(stt_ref[...], axis=0)
    st2 = jnp.sum(st2_ref[...], axis=0)
    for g in range(tp_ref.shape[0]):
        yt = _bn2d(tp_ref[g], stt, tg_ref[...], tb_ref[...])
        rpre = _mm(wrs_ref[...], x2_ref[g]) + brs_ref[...]
        yr = _bn2d(rpre, st2[:, 4:6], rg_ref[...], rb_ref[...])
        out = jnp.maximum(yt + yr, 0.0)
        xc_ref[g] = jnp.transpose(jnp.mean(out, axis=1, keepdims=True))
        xr_ref[g] = jnp.mean(out, axis=0, keepdims=True)


def _col(v):
    return v.reshape(-1, 1)


def _row(v):
    return v.reshape(1, -1)


def _layer_pre(p, c_in, inter, c_out):
    wt = jnp.transpose(p['tcn_w'][:, :, :, 0], (2, 0, 1))
    return dict(
        pa=p['PA'].reshape(3 * _V, _V),
        wa=p['conv_a_w'].reshape(3 * inter, c_in),
        wb=p['conv_b_w'].reshape(3 * inter, c_in),
        wdn=p['down_w'], bdn=_col(p['down_b']),
        wrs=p['res_w'].reshape(c_out, c_in), brs=_col(p['res_b']),
        ba=p['conv_a_b'].reshape(3 * inter, 1),
        bb=p['conv_b_b'].reshape(3 * inter, 1),
        wd3=p['conv_d_w'], bd3=p['conv_d_b'],
        al=p['alpha'].reshape(1, 1),
        gg=_col(p['gcn_bn_g']), gb=_col(p['gcn_bn_b']),
        dg=_col(p['down_bn_g']), db=_col(p['down_bn_b']),
        tg=_col(p['tcn_bn_g']), tb=_col(p['tcn_bn_b']),
        rg=_col(p['res_bn_g']), rb=_col(p['res_bn_b']),
        sat=p['sa_w'].reshape(-1, 127), sab=p['sa_b'].reshape(1, 1),
        taw=p['ta_w'].reshape(-1, 9), tab=p['ta_b'].reshape(1, 1),
        f1w=p['fc1_w'], f1b=_col(p['fc1_b']),
        f2w=p['fc2_w'], f2b=_col(p['fc2_b']),
        wtc=wt.reshape(9 * c_out, c_out), tcb=_col(p['tcn_b']),
    )


def _full(a):
    nd = a.ndim
    return pl.BlockSpec(a.shape, lambda n: (0,) * nd)


def _per_n(shape, g=1):
    return pl.BlockSpec((g,) + shape, lambda n: (n, 0, 0))


def _act(c):
    return jax.ShapeDtypeStruct((_B, c, _TV), _F32)


def _stats(c, w):
    return jax.ShapeDtypeStruct((_B, c, w), _F32)


def kernel(touch_input, params):
    l1 = _layer_pre(params['layer1'], 1, 8, 32)
    l2 = _layer_pre(params['layer2'], 32, 16, 64)
    x = touch_input.reshape(_B, 1, _TV)

    gcn_keys = ('pa', 'wa', 'wb', 'wdn', 'wrs', 'ba', 'bb', 'bdn', 'brs',
                'wd3', 'bd3', 'al')
    k1_w = [l1[k] for k in gcn_keys]
    ypre1, st1 = pl.pallas_call(
        _k1_body, grid=(_B // 4,),
        in_specs=[_per_n((1, _TV), 4)] + [_full(a) for a in k1_w],
        out_specs=[_per_n((32, _TV), 4), _per_n((32, 6), 4)],
        out_shape=[_act(32), _stats(32, 6)],
    )(x, *k1_w)

    k2_w = [l1[k] for k in ('wdn', 'bdn')]
    k2_w2 = [l1[k] for k in ('gg', 'gb', 'dg', 'db', 'sat', 'sab', 'taw',
                             'tab', 'f1w', 'f1b', 'f2w', 'f2b', 'wtc',
                             'tcb')]
    tcn1, stt1 = pl.pallas_call(
        _k2_body, grid=(_B // 4,),
        in_specs=[_per_n((32, _TV), 4), _per_n((1, _TV), 4)]
                 + [_full(a) for a in k2_w] + [_full(st1)]
                 + [_full(a) for a in k2_w2],
        out_specs=[_per_n((32, _TV), 4), _per_n((32, 2), 4)],
        out_shape=[_act(32), _stats(32, 2)],
    )(ypre1, x, *k2_w, st1, *k2_w2)

    k3_w = ([l1[k] for k in ('tg', 'tb', 'rg', 'rb', 'wrs', 'brs')]
            + [l2[k] for k in gcn_keys])
    ypre2, x2, st2 = pl.pallas_call(
        _k3_body, grid=(_B // 2,),
        in_specs=[_per_n((32, _TV), 2), _per_n((1, _TV), 2)]
                 + [_full(stt1), _full(st1)] + [_full(a) for a in k3_w],
        out_specs=[_per_n((64, _TV), 2), _per_n((32, _TV), 2),
                   _per_n((64, 6), 2)],
        out_shape=[_act(64), _act(32), _stats(64, 6)],
    )(tcn1, x, stt1, st1, *k3_w)

    k4_w = [l2[k] for k in ('wdn', 'bdn')]
    k4_w2 = [l2[k] for k in ('gg', 'gb', 'dg', 'db', 'sat', 'sab', 'taw',
                             'tab', 'f1w', 'f1b', 'f2w', 'f2b', 'wtc',
                             'tcb')]
    tcn2, stt2 = pl.pallas_call(
        _k4_body, grid=(_B // 4,),
        in_specs=[_per_n((64, _TV), 4), _per_n((32, _TV), 4)]
                 + [_full(a) for a in k4_w] + [_full(st2)]
                 + [_full(a) for a in k4_w2],
        out_specs=[_per_n((64, _TV), 4), _per_n((64, 2), 4)],
        out_shape=[_act(64), _stats(64, 2)],
    )(ypre2, x2, *k4_w, st2, *k4_w2)

    k5_w = [l2[k] for k in ('tg', 'tb', 'rg', 'rb', 'wrs', 'brs')]
    xc, xr = pl.pallas_call(
        _k5_body, grid=(_B // 4,),
        in_specs=[_per_n((64, _TV), 4), _per_n((32, _TV), 4)]
                 + [_full(stt2), _full(st2)] + [_full(a) for a in k5_w],
        out_specs=[_per_n((1, 64), 4), _per_n((1, _TV), 4)],
        out_shape=[jax.ShapeDtypeStruct((_B, 1, 64), _F32),
                   jax.ShapeDtypeStruct((_B, 1, _TV), _F32)],
    )(tcn2, x2, stt2, st2, *k5_w)

    return (xc.reshape(_B, 64), xr.reshape(_B, _T, _V))


# speedup vs baseline: 1.0053x; 1.0053x over previous
import jax
import jax.numpy as jnp
from jax.experimental import pallas as pl
from jax.experimental.pallas import tpu as pltpu

_V = 128
_T = 64
_B = 16
_TV = _T * _V
_NTV = float(_B * _TV)
_EPS = 1e-5
_PREC = jax.lax.Precision.DEFAULT
_F32 = jnp.float32


def _mm(a, b, dn=(((1,), (0,)), ((), ()))):
    return jax.lax.dot_general(a, b, dn, precision=_PREC,
                               preferred_element_type=_F32)


def _sumsq(v):
    return jnp.concatenate([jnp.sum(v, axis=1, keepdims=True),
                            jnp.sum(v * v, axis=1, keepdims=True)], axis=1)


def _bn2d(v, st2, g_col, b_col):
    mean = st2[:, 0:1] / _NTV
    var = st2[:, 1:2] / _NTV - mean * mean
    scale = g_col * jax.lax.rsqrt(var + _EPS)
    return v * scale + (b_col - mean * scale)


def _gcn_prep(c_in, inter, c_out, pa_ref, wa_ref, wb_ref, wdn_ref, wrs_ref,
              ba_ref, bb_ref, bdn_ref, brs_ref, wd3_ref, bd3_ref, al_ref):
    paw = pa_ref[...]
    pa_cat = jnp.concatenate([paw[i * _V:(i + 1) * _V, :]
                              for i in range(3)], axis=1)
    wfront = jnp.concatenate([wa_ref[...], wb_ref[...], wdn_ref[...],
                              wrs_ref[...]], axis=0)
    bfront = jnp.concatenate([ba_ref[...], bb_ref[...], bdn_ref[...],
                              brs_ref[...]], axis=0)
    wd3 = wd3_ref[...]
    wd = jnp.concatenate([wd3[i] for i in range(3)],
                         axis=0 if c_in == 1 else 1)
    bd_sum = jnp.transpose(jnp.sum(bd3_ref[...], axis=0, keepdims=True))
    return pa_cat, wfront, bfront, wd, bd_sum, al_ref[...]


def _gcn_stage(x2d, c_in, inter, c_out, pa_cat, wfront, bfront, wd, bd_sum,
               alpha):
    si = 6 * inter
    if c_in == 1:
        fr = wfront * x2d + bfront
    else:
        fr = _mm(wfront, x2d) + bfront
    ab = fr[:si, :]
    d = fr[si:si + c_out, :]
    r = fr[si + c_out:, :]
    ms = []
    for i in range(3):
        a_i = ab[i * inter:(i + 1) * inter, :]
        b_i = ab[(3 + i) * inter:(4 + i) * inter, :]
        a_i = a_i.reshape(inter, _T, _V).reshape(inter * _T, _V)
        b_i = b_i.reshape(inter, _T, _V).reshape(inter * _T, _V)
        m = _mm(a_i, b_i, (((0,), (0,)), ((), ())))
        ms.append(m / float(inter * _T))
    a_cat = pa_cat + jnp.tanh(jnp.concatenate(ms, axis=1)) * alpha
    if c_in == 1:
        x_tv = x2d.reshape(_T, _V)
        z_cat = _mm(x_tv, a_cat)
        y = bd_sum + jnp.zeros((c_out, _TV), _F32)
        for i in range(3):
            z2d = z_cat[:, i * _V:(i + 1) * _V].reshape(1, _TV)
            y = y + wd[i * c_out:(i + 1) * c_out, :] * z2d
    else:
        x_ctv = x2d.reshape(c_in, _T, _V).reshape(c_in * _T, _V)
        z_cat = _mm(x_ctv, a_cat)
        zr = jnp.concatenate(
            [z_cat[:, i * _V:(i + 1) * _V].reshape(c_in, _T, _V)
             .reshape(c_in, _TV) for i in range(3)], axis=0)
        y = _mm(wd, zr) + bd_sum
    return y, d, r


def _k1_body(x_ref, *refs):
    (y_ref, st_ref), w_refs = refs[-2:], refs[:-2]
    prep = _gcn_prep(1, 8, 32, *w_refs)
    for g in range(x_ref.shape[0]):
        y, d, r = _gcn_stage(x_ref[g], 1, 8, 32, *prep)
        y_ref[g] = y
        st_ref[g] = jnp.concatenate([_sumsq(y), _sumsq(d), _sumsq(r)],
                                    axis=1)


def _k3_body(tp_ref, x_ref, stt_ref, st1_ref, tg_ref, tb_ref, rg_ref,
             rb_ref, wrs_ref, brs_ref, *refs):
    (y_ref, x2_ref, st_ref), w_refs = refs[-3:], refs[:-3]
    prep = _gcn_prep(32, 16, 64, *w_refs)
    stt = jnp.sum(stt_ref[...], axis=0)
    st1 = jnp.sum(st1_ref[...], axis=0)
    for g in range(tp_ref.shape[0]):
        yt = _bn2d(tp_ref[g], stt, tg_ref[...], tb_ref[...])
        rpre = wrs_ref[...] * x_ref[g] + brs_ref[...]
        yr = _bn2d(rpre, st1[:, 4:6], rg_ref[...], rb_ref[...])
        x2d = jnp.maximum(yt + yr, 0.0)
        y, d, r = _gcn_stage(x2d, 32, 16, 64, *prep)
        y_ref[g] = y
        x2_ref[g] = x2d
        st_ref[g] = jnp.concatenate([_sumsq(y), _sumsq(d), _sumsq(r)],
                                    axis=1)


def _attn_tcn(c, c_half, g, dpre, y_ref, st_ref, gg_ref, gb_ref, dg_ref,
              db_ref, sat_ref, sab_ref, taw_ref, tab_ref, f1w_ref, f1b_ref,
              f2w_ref, f2b_ref, wt_ref, tb_ref, o_ref, ost_ref):
    st = jnp.sum(st_ref[...], axis=0)
    yb = _bn2d(y_ref[g], st[:, 0:2], gg_ref[...], gb_ref[...])
    db = _bn2d(dpre, st[:, 2:4], dg_ref[...], db_ref[...])
    y = jnp.maximum(yb + db, 0.0)
    y3 = y.reshape(c, _T, _V)
    se = jnp.mean(y3, axis=1)
    p_mat = _mm(sat_ref[...], se, (((0,), (0,)), ((), ())))
    p_big = jnp.concatenate([jnp.zeros((127, 64), _F32), p_mat,
                             jnp.zeros((127, 64), _F32)], axis=1)
    p_big = jnp.concatenate([p_big, jnp.zeros((1, 256), _F32)], axis=0)
    pr = p_big.reshape(16, 8, 256)
    b = pr[:, 0, 0:249]
    for k2 in range(1, 8):
        b = b + pr[:, k2, k2:k2 + 249]
    s1 = jnp.zeros((1, _V), _F32)
    for k1 in range(16):
        s1 = s1 + b[k1:k1 + 1, 8 * k1 + 1: 8 * k1 + 1 + _V]
    s1 = jax.nn.sigmoid(s1 + sab_ref[...])
    se2 = jnp.mean(y3 * (1.0 + s1.reshape(1, 1, _V)), axis=2)
    se2p = jnp.concatenate([jnp.zeros((c, 4), _F32), se2,
                            jnp.zeros((c, 4), _F32)], axis=1)
    taw = taw_ref[...]
    s1t = jnp.zeros((1, _T), _F32) + tab_ref[...]
    for k in range(9):
        s1t = s1t + _mm(taw[:, k:k + 1], se2p[:, k:k + _T],
                        (((0,), (0,)), ((), ())))
    s1t = jax.nn.sigmoid(s1t)
    se3 = jnp.mean(se2 * (1.0 + s1t), axis=1, keepdims=True)
    h = jnp.maximum(_mm(f1w_ref[...], se3) + f1b_ref[...], 0.0)
    s2 = jax.nn.sigmoid(_mm(f2w_ref[...], h) + f2b_ref[...])
    fac = (1.0 + jnp.transpose(s1t)) * (1.0 + s1)
    y = (y3 * fac).reshape(c, _TV)
    yp = jnp.concatenate([jnp.zeros((c, 4 * _V), _F32), y,
                          jnp.zeros((c, 4 * _V), _F32)], axis=1)
    wt = wt_ref[...] * (1.0 + jnp.transpose(s2))
    t = tb_ref[...] + jnp.zeros((c, _TV), _F32)
    for k in range(9):
        t = t + _mm(wt[k * c:(k + 1) * c, :], yp[:, k * _V:k * _V + _TV])
    o_ref[g] = t
    ost_ref[g] = _sumsq(t)


def _k2_body(y_ref, x_ref, wdn_ref, bdn_ref, *refs):
    for g in range(y_ref.shape[0]):
        dpre = wdn_ref[...] * x_ref[g] + bdn_ref[...]
        _attn_tcn(32, 16, g, dpre, y_ref, *refs)


def _k4_body(y_ref, x2_ref, wdn_ref, bdn_ref, *refs):
    for g in range(y_ref.shape[0]):
        dpre = _mm(wdn_ref[...], x2_ref[g]) + bdn_ref[...]
        _attn_tcn(64, 32, g, dpre, y_ref, *refs)


def _k5_body(tp_ref, x2_ref, stt_ref, st2_ref, tg_ref, tb_ref, rg_ref,
             rb_ref, wrs_ref, brs_ref, xc_ref, xr_ref):
    stt = jnp.sum(stt_ref[...], axis=0)
    st2 = jnp.sum(st2_ref[...], axis=0)
    for g in range(tp_ref.shape[0]):
        yt = _bn2d(tp_ref[g], stt, tg_ref[...], tb_ref[...])
        rpre = _mm(wrs_ref[...], x2_ref[g]) + brs_ref[...]
        yr = _bn2d(rpre, st2[:, 4:6], rg_ref[...], rb_ref[...])
        out = jnp.maximum(yt + yr, 0.0)
        xc_ref[g] = jnp.transpose(jnp.mean(out, axis=1, keepdims=True))
        xr_ref[g] = jnp.mean(out, axis=0, keepdims=True)


def _col(v):
    return v.reshape(-1, 1)


def _row(v):
    return v.reshape(1, -1)


def _layer_pre(p, c_in, inter, c_out):
    wt = jnp.transpose(p['tcn_w'][:, :, :, 0], (2, 0, 1))
    return dict(
        pa=p['PA'].reshape(3 * _V, _V),
        wa=p['conv_a_w'].reshape(3 * inter, c_in),
        wb=p['conv_b_w'].reshape(3 * inter, c_in),
        wdn=p['down_w'], bdn=_col(p['down_b']),
        wrs=p['res_w'].reshape(c_out, c_in), brs=_col(p['res_b']),
        ba=p['conv_a_b'].reshape(3 * inter, 1),
        bb=p['conv_b_b'].reshape(3 * inter, 1),
        wd3=p['conv_d_w'], bd3=p['conv_d_b'],
        al=p['alpha'].reshape(1, 1),
        gg=_col(p['gcn_bn_g']), gb=_col(p['gcn_bn_b']),
        dg=_col(p['down_bn_g']), db=_col(p['down_bn_b']),
        tg=_col(p['tcn_bn_g']), tb=_col(p['tcn_bn_b']),
        rg=_col(p['res_bn_g']), rb=_col(p['res_bn_b']),
        sat=p['sa_w'].reshape(-1, 127), sab=p['sa_b'].reshape(1, 1),
        taw=p['ta_w'].reshape(-1, 9), tab=p['ta_b'].reshape(1, 1),
        f1w=p['fc1_w'], f1b=_col(p['fc1_b']),
        f2w=p['fc2_w'], f2b=_col(p['fc2_b']),
        wtc=wt.reshape(9 * c_out, c_out), tcb=_col(p['tcn_b']),
    )


def _full(a):
    nd = a.ndim
    return pl.BlockSpec(a.shape, lambda n: (0,) * nd)


def _per_n(shape, g=1):
    return pl.BlockSpec((g,) + shape, lambda n: (n, 0, 0))


def _act(c):
    return jax.ShapeDtypeStruct((_B, c, _TV), _F32)


def _stats(c, w):
    return jax.ShapeDtypeStruct((_B, c, w), _F32)


def kernel(touch_input, params):
    l1 = _layer_pre(params['layer1'], 1, 8, 32)
    l2 = _layer_pre(params['layer2'], 32, 16, 64)
    x = touch_input.reshape(_B, 1, _TV)

    gcn_keys = ('pa', 'wa', 'wb', 'wdn', 'wrs', 'ba', 'bb', 'bdn', 'brs',
                'wd3', 'bd3', 'al')
    k1_w = [l1[k] for k in gcn_keys]
    ypre1, st1 = pl.pallas_call(
        _k1_body, grid=(_B // 4,),
        in_specs=[_per_n((1, _TV), 4)] + [_full(a) for a in k1_w],
        out_specs=[_per_n((32, _TV), 4), _per_n((32, 6), 4)],
        out_shape=[_act(32), _stats(32, 6)],
    )(x, *k1_w)

    k2_w = [l1[k] for k in ('wdn', 'bdn')]
    k2_w2 = [l1[k] for k in ('gg', 'gb', 'dg', 'db', 'sat', 'sab', 'taw',
                             'tab', 'f1w', 'f1b', 'f2w', 'f2b', 'wtc',
                             'tcb')]
    tcn1, stt1 = pl.pallas_call(
        _k2_body, grid=(_B // 4,),
        in_specs=[_per_n((32, _TV), 4), _per_n((1, _TV), 4)]
                 + [_full(a) for a in k2_w] + [_full(st1)]
                 + [_full(a) for a in k2_w2],
        out_specs=[_per_n((32, _TV), 4), _per_n((32, 2), 4)],
        out_shape=[_act(32), _stats(32, 2)],
    )(ypre1, x, *k2_w, st1, *k2_w2)

    k3_w = ([l1[k] for k in ('tg', 'tb', 'rg', 'rb', 'wrs', 'brs')]
            + [l2[k] for k in gcn_keys])
    ypre2, x2, st2 = pl.pallas_call(
        _k3_body, grid=(_B // 2,),
        in_specs=[_per_n((32, _TV), 2), _per_n((1, _TV), 2)]
                 + [_full(stt1), _full(st1)] + [_full(a) for a in k3_w],
        out_specs=[_per_n((64, _TV), 2), _per_n((32, _TV), 2),
                   _per_n((64, 6), 2)],
        out_shape=[_act(64), _act(32), _stats(64, 6)],
    )(tcn1, x, stt1, st1, *k3_w)

    k4_w = [l2[k] for k in ('wdn', 'bdn')]
    k4_w2 = [l2[k] for k in ('gg', 'gb', 'dg', 'db', 'sat', 'sab', 'taw',
                             'tab', 'f1w', 'f1b', 'f2w', 'f2b', 'wtc',
                             'tcb')]
    tcn2, stt2 = pl.pallas_call(
        _k4_body, grid=(_B // 2,),
        in_specs=[_per_n((64, _TV), 2), _per_n((32, _TV), 2)]
                 + [_full(a) for a in k4_w] + [_full(st2)]
                 + [_full(a) for a in k4_w2],
        out_specs=[_per_n((64, _TV), 2), _per_n((64, 2), 2)],
        out_shape=[_act(64), _stats(64, 2)],
    )(ypre2, x2, *k4_w, st2, *k4_w2)

    k5_w = [l2[k] for k in ('tg', 'tb', 'rg', 'rb', 'wrs', 'brs')]
    xc, xr = pl.pallas_call(
        _k5_body, grid=(_B // 4,),
        in_specs=[_per_n((64, _TV), 4), _per_n((32, _TV), 4)]
                 + [_full(stt2), _full(st2)] + [_full(a) for a in k5_w],
        out_specs=[_per_n((1, 64), 4), _per_n((1, _TV), 4)],
        out_shape=[jax.ShapeDtypeStruct((_B, 1, 64), _F32),
                   jax.ShapeDtypeStruct((_B, 1, _TV), _F32)],
    )(tcn2, x2, stt2, st2, *k5_w)

    return (xc.reshape(_B, 64), xr.reshape(_B, _T, _V))
